# Initial kernel scaffold; baseline (speedup 1.0000x reference)
import functools

import jax
import jax.numpy as jnp
from jax import lax
from jax.experimental import pallas as pl
from jax.experimental.pallas import tpu as pltpu
from jax.experimental.pallas import tpu_sc as plsc

ND = 100000
NT = 100000
E = 3200000
C = 16

NC = 2
NS = 16
NW = NC * NS
CHUNK = 128
KCH = 784
EPT = KCH * CHUNK
E_PAD = NW * EPT
ROWS_PT = 6256
NT_PAD = NS * ROWS_PT

EB_LANE = 512
RB = E_PAD // EB_LANE
EB_ROWS = 128
GRID_B = RB // EB_ROWS

BN = 12500
GRID_N = NT // BN

NEG_BIG = -1e30


def _pre_body(dx, tx, Ws, Wd, asv, adv, Wr, b, xs_o, as_o, ad_o, res_o):
    dxv = dx[...]
    txv = tx[...]
    Wsv = Ws[...]
    xs = dxv[:, 0:1] * Wsv[0:1, :]
    for k in range(1, 5):
        xs = xs + dxv[:, k:k + 1] * Wsv[k:k + 1, :]
    xs_o[...] = xs
    as_o[...] = jnp.sum(xs * asv[...][None, :], axis=1)
    wdv = jnp.sum(Wd[...] * adv[...][None, :], axis=1)
    ad = txv[:, 0] * wdv[0]
    for k in range(1, 12):
        ad = ad + txv[:, k] * wdv[k]
    ad_o[...] = ad
    Wrv = Wr[...]
    res = txv[:, 0:1] * Wrv[0:1, :]
    for k in range(1, 12):
        res = res + txv[:, k:k + 1] * Wrv[k:k + 1, :]
    res_o[...] = res + b[...][None, :]


def _pre_call(data_x, tasks_x, W_src, W_dst, att_src_v, att_dst_v, res_W, bias):
    return pl.pallas_call(
        _pre_body,
        grid=(GRID_N,),
        in_specs=[
            pl.BlockSpec((BN, 5), lambda i: (i, 0)),
            pl.BlockSpec((BN, 12), lambda i: (i, 0)),
            pl.BlockSpec((5, C), lambda i: (0, 0)),
            pl.BlockSpec((12, C), lambda i: (0, 0)),
            pl.BlockSpec((C,), lambda i: (0,)),
            pl.BlockSpec((C,), lambda i: (0,)),
            pl.BlockSpec((12, C), lambda i: (0, 0)),
            pl.BlockSpec((C,), lambda i: (0,)),
        ],
        out_specs=[
            pl.BlockSpec((BN, C), lambda i: (i, 0)),
            pl.BlockSpec((BN,), lambda i: (i,)),
            pl.BlockSpec((BN,), lambda i: (i,)),
            pl.BlockSpec((BN, C), lambda i: (i, 0)),
        ],
        out_shape=[
            jax.ShapeDtypeStruct((ND, C), jnp.float32),
            jax.ShapeDtypeStruct((ND,), jnp.float32),
            jax.ShapeDtypeStruct((NT,), jnp.float32),
            jax.ShapeDtypeStruct((NT, C), jnp.float32),
        ],
    )(data_x, tasks_x, W_src, W_dst, att_src_v, att_dst_v, res_W, bias)


def _edge_body(eat, We, aev, ae_o):
    w3 = jnp.sum(We[...] * aev[...][None, :], axis=1)
    b = eat[...]
    val = b[0] * w3[0] + b[1] * w3[1] + b[2] * w3[2]
    i = pl.program_id(0)
    rows = lax.broadcasted_iota(jnp.int32, (EB_ROWS, EB_LANE), 0)
    cols = lax.broadcasted_iota(jnp.int32, (EB_ROWS, EB_LANE), 1)
    flat = (i * EB_ROWS + rows) * EB_LANE + cols
    ae_o[...] = jnp.where(flat < E, val, NEG_BIG)


def _edge_call(eat3, W_edge, att_edge_v):
    return pl.pallas_call(
        _edge_body,
        grid=(GRID_B,),
        in_specs=[
            pl.BlockSpec((3, EB_ROWS, EB_LANE), lambda i: (0, i, 0)),
            pl.BlockSpec((3, C), lambda i: (0, 0)),
            pl.BlockSpec((C,), lambda i: (0,)),
        ],
        out_specs=pl.BlockSpec((EB_ROWS, EB_LANE), lambda i: (i, 0)),
        out_shape=jax.ShapeDtypeStruct((RB, EB_LANE), jnp.float32),
    )(eat3, W_edge, att_edge_v)


def _sc_body(src_h, dst_h, ae_h, xsrc_h, asrc_h, adst_h, z16_h, z1_h,
             acc_o, s_o,
             acc_sh, s_sh, srcv, dstv, aev, asv, adv, xs, msg, exb, sem):
    ci = lax.axis_index("c")
    si = lax.axis_index("s")
    wid = ci * NS + si
    r0 = si * ROWS_PT
    pltpu.sync_copy(z16_h, acc_sh.at[pl.ds(r0, ROWS_PT)])
    pltpu.sync_copy(z1_h, s_sh.at[pl.ds(r0, ROWS_PT)])
    plsc.subcore_barrier()

    base0 = wid * EPT

    def chunk(k, carry):
        base = base0 + k * CHUNK
        pltpu.sync_copy(src_h.at[pl.ds(base, CHUNK)], srcv)
        pltpu.sync_copy(dst_h.at[pl.ds(base, CHUNK)], dstv)
        pltpu.sync_copy(ae_h.at[pl.ds(base, CHUNK)], aev)
        cp1 = pltpu.async_copy(xsrc_h.at[srcv], xs, sem)
        cp2 = pltpu.async_copy(asrc_h.at[srcv], asv, sem)
        cp3 = pltpu.async_copy(adst_h.at[dstv], adv, sem)
        cp1.wait()
        cp2.wait()
        cp3.wait()
        for j in range(CHUNK // 16):
            sl = pl.ds(j * 16, 16)
            al = asv[sl] + adv[sl] + aev[sl]
            al = jnp.maximum(al, al * 0.2)
            exb[sl] = jnp.exp(al)
        for i in range(CHUNK):
            msg[i, :] = xs[i, :] * exb[i]
        pltpu.sync_copy(exb, s_sh.at[dstv], add=True)
        pltpu.sync_copy(msg, acc_sh.at[dstv], add=True)
        return carry

    lax.fori_loop(0, KCH, chunk, 0)
    plsc.subcore_barrier()
    pltpu.sync_copy(acc_sh.at[pl.ds(r0, ROWS_PT)],
                    acc_o.at[ci, pl.ds(r0, ROWS_PT)])
    pltpu.sync_copy(s_sh.at[pl.ds(r0, ROWS_PT)],
                    s_o.at[ci, pl.ds(r0, ROWS_PT)])


_sc_call = functools.partial(
    pl.kernel,
    out_type=[
        jax.ShapeDtypeStruct((NC, NT_PAD, C), jnp.float32),
        jax.ShapeDtypeStruct((NC, NT_PAD), jnp.float32),
    ],
    mesh=plsc.VectorSubcoreMesh(core_axis_name="c", subcore_axis_name="s"),
    scratch_types=[
        pltpu.VMEM_SHARED((NT_PAD, C), jnp.float32),
        pltpu.VMEM_SHARED((NT_PAD,), jnp.float32),
        pltpu.VMEM((CHUNK,), jnp.int32),
        pltpu.VMEM((CHUNK,), jnp.int32),
        pltpu.VMEM((CHUNK,), jnp.float32),
        pltpu.VMEM((CHUNK,), jnp.float32),
        pltpu.VMEM((CHUNK,), jnp.float32),
        pltpu.VMEM((CHUNK, C), jnp.float32),
        pltpu.VMEM((CHUNK, C), jnp.float32),
        pltpu.VMEM((CHUNK,), jnp.float32),
        pltpu.SemaphoreType.DMA,
    ],
)(_sc_body)


def _post_body(acc, s, res, g, b, out_o):
    accv = acc[...]
    sv = s[...]
    a = accv[0] + accv[1]
    ssum = sv[0] + sv[1]
    out = a / (ssum[:, None] + 1e-16) + res[...]
    mu = jnp.mean(out, axis=1, keepdims=True)
    d = out - mu
    var = jnp.mean(d * d, axis=1, keepdims=True)
    y = d * lax.rsqrt(var + 1e-5) * g[...][None, :] + b[...][None, :]
    out_o[...] = jnp.maximum(y, 0.01 * y)


def _post_call(acc, s, res, ln_gamma, ln_beta):
    return pl.pallas_call(
        _post_body,
        grid=(GRID_N,),
        in_specs=[
            pl.BlockSpec((NC, BN, C), lambda i: (0, i, 0)),
            pl.BlockSpec((NC, BN), lambda i: (0, i)),
            pl.BlockSpec((BN, C), lambda i: (i, 0)),
            pl.BlockSpec((C,), lambda i: (0,)),
            pl.BlockSpec((C,), lambda i: (0,)),
        ],
        out_specs=pl.BlockSpec((BN, C), lambda i: (i, 0)),
        out_shape=jax.ShapeDtypeStruct((NT, C), jnp.float32),
    )(acc, s, res, ln_gamma, ln_beta)


def kernel(data_x, tasks_x, edge_index, edge_attr, W_src, W_dst, att_src,
           att_dst, W_edge, att_edge, res_W, bias, ln_gamma, ln_beta):
    att_src_v = att_src.reshape(C)
    att_dst_v = att_dst.reshape(C)
    att_edge_v = att_edge.reshape(C)

    xsrc, asrc, adst, res = _pre_call(
        data_x, tasks_x, W_src, W_dst, att_src_v, att_dst_v, res_W, bias)

    ei = jnp.pad(edge_index, ((0, 0), (0, E_PAD - E)))
    src = ei[0]
    dst = ei[1]
    eat3 = jnp.pad(edge_attr, ((0, E_PAD - E), (0, 0))).T.reshape(3, RB, EB_LANE)
    ae = _edge_call(eat3, W_edge, att_edge_v).reshape(E_PAD)

    z16 = jnp.zeros((ROWS_PT, C), jnp.float32)
    z1 = jnp.zeros((ROWS_PT,), jnp.float32)
    acc, s = _sc_call(src, dst, ae, xsrc, asrc, adst, z16, z1)

    return _post_call(acc, s, res, ln_gamma, ln_beta)


# speedup vs baseline: 46.0614x; 46.0614x over previous
import functools

import jax
import jax.numpy as jnp
from jax import lax
from jax.experimental import pallas as pl
from jax.experimental.pallas import tpu as pltpu
from jax.experimental.pallas import tpu_sc as plsc

ND = 100000
NT = 100000
E = 3200000
C = 16

NC = 2
NS = 16
NW = NC * NS
CHUNK = 128
KCH = 784
EPT = KCH * CHUNK
E_PAD = NW * EPT
ROWS_PT = 6272
NT_PAD = NS * ROWS_PT
ZC = 784
NZ = ROWS_PT // ZC
GCH = NS * NZ

EB_LANE = 512
RB = E_PAD // EB_LANE
EB_ROWS = 128
GRID_B = RB // EB_ROWS

BN = 2000
GRID_N = NT // BN

NEG_BIG = -1e30


def _pre_body(dx, tx, Ws, Wd, asv, adv, Wr, b, xs_o, as_o, ad_o, res_o):
    dxv = dx[...]
    txv = tx[...]
    Wsv = Ws[...]
    xs = dxv[:, 0:1] * Wsv[0:1, :]
    for k in range(1, 5):
        xs = xs + dxv[:, k:k + 1] * Wsv[k:k + 1, :]
    xs_o[...] = xs
    as_o[...] = jnp.sum(xs * asv[...][None, :], axis=1).reshape(1, 1, BN)
    wdv = jnp.sum(Wd[...] * adv[...][None, :], axis=1)
    ad = txv[:, 0] * wdv[0]
    for k in range(1, 12):
        ad = ad + txv[:, k] * wdv[k]
    ad_o[...] = ad.reshape(1, 1, BN)
    Wrv = Wr[...]
    res = txv[:, 0:1] * Wrv[0:1, :]
    for k in range(1, 12):
        res = res + txv[:, k:k + 1] * Wrv[k:k + 1, :]
    res_o[...] = res + b[...][None, :]


def _pre_call(data_x, tasks_x, W_src, W_dst, att_src_v, att_dst_v, res_W, bias):
    return pl.pallas_call(
        _pre_body,
        grid=(GRID_N,),
        in_specs=[
            pl.BlockSpec((BN, 5), lambda i: (i, 0)),
            pl.BlockSpec((BN, 12), lambda i: (i, 0)),
            pl.BlockSpec((5, C), lambda i: (0, 0)),
            pl.BlockSpec((12, C), lambda i: (0, 0)),
            pl.BlockSpec((C,), lambda i: (0,)),
            pl.BlockSpec((C,), lambda i: (0,)),
            pl.BlockSpec((12, C), lambda i: (0, 0)),
            pl.BlockSpec((C,), lambda i: (0,)),
        ],
        out_specs=[
            pl.BlockSpec((BN, C), lambda i: (i, 0)),
            pl.BlockSpec((1, 1, BN), lambda i: (i, 0, 0)),
            pl.BlockSpec((1, 1, BN), lambda i: (i, 0, 0)),
            pl.BlockSpec((BN, C), lambda i: (i, 0)),
        ],
        out_shape=[
            jax.ShapeDtypeStruct((ND, C), jnp.float32),
            jax.ShapeDtypeStruct((GRID_N, 1, BN), jnp.float32),
            jax.ShapeDtypeStruct((GRID_N, 1, BN), jnp.float32),
            jax.ShapeDtypeStruct((NT, C), jnp.float32),
        ],
    )(data_x, tasks_x, W_src, W_dst, att_src_v, att_dst_v, res_W, bias)


def _edge_body(eat, We, aev, ae_o):
    w3 = jnp.sum(We[...] * aev[...][None, :], axis=1)
    b = eat[...]
    val = b[0] * w3[0] + b[1] * w3[1] + b[2] * w3[2]
    i = pl.program_id(0)
    rows = lax.broadcasted_iota(jnp.int32, (EB_ROWS, EB_LANE), 0)
    cols = lax.broadcasted_iota(jnp.int32, (EB_ROWS, EB_LANE), 1)
    flat = (i * EB_ROWS + rows) * EB_LANE + cols
    ae_o[...] = jnp.where(flat < E, val, NEG_BIG)


def _edge_call(eat3, W_edge, att_edge_v):
    return pl.pallas_call(
        _edge_body,
        grid=(GRID_B,),
        in_specs=[
            pl.BlockSpec((3, EB_ROWS, EB_LANE), lambda i: (0, i, 0)),
            pl.BlockSpec((3, C), lambda i: (0, 0)),
            pl.BlockSpec((C,), lambda i: (0,)),
        ],
        out_specs=pl.BlockSpec((EB_ROWS, EB_LANE), lambda i: (i, 0)),
        out_shape=jax.ShapeDtypeStruct((RB, EB_LANE), jnp.float32),
    )(eat3, W_edge, att_edge_v)


def _sc_body(src_h, dst_h, ae_h, xsrc_h, asrc_h, adst_h, z16_h, z1_h,
             acc_o, s_o,
             acc_sh, s_sh, srcv, dstv, aev, asv, adv, xs, msg, exb,
             zb16, zb1, sem):
    ci = lax.axis_index("c")
    si = lax.axis_index("s")
    wid = ci * NS + si
    r0 = si * ROWS_PT
    pltpu.sync_copy(z16_h, zb16)
    pltpu.sync_copy(z1_h, zb1)
    for t in range(NZ):
        row = r0 + t * ZC
        pltpu.sync_copy(zb16, acc_sh.at[pl.ds(row, ZC)])
        pltpu.sync_copy(zb1, s_sh.at[pl.ds(row, ZC)])
    plsc.subcore_barrier()

    base0 = wid * EPT

    def chunk(k, carry):
        base = base0 + k * CHUNK
        pltpu.sync_copy(src_h.at[pl.ds(base, CHUNK)], srcv)
        pltpu.sync_copy(dst_h.at[pl.ds(base, CHUNK)], dstv)
        pltpu.sync_copy(ae_h.at[pl.ds(base, CHUNK)], aev)
        cp1 = pltpu.async_copy(xsrc_h.at[srcv], xs, sem)
        cp2 = pltpu.async_copy(asrc_h.at[srcv], asv, sem)
        cp3 = pltpu.async_copy(adst_h.at[dstv], adv, sem)
        cp1.wait()
        cp2.wait()
        cp3.wait()
        for j in range(CHUNK // 16):
            sl = pl.ds(j * 16, 16)
            al = asv[sl] + adv[sl] + aev[sl]
            al = jnp.maximum(al, al * 0.2)
            ex = jnp.exp(al)
            exb[sl] = ex
            for l in range(16):
                i = j * 16 + l
                msg[i, :] = xs[i, :] * ex[l]
        pltpu.sync_copy(exb, s_sh.at[dstv], add=True)
        pltpu.sync_copy(msg, acc_sh.at[dstv], add=True)
        return carry

    lax.fori_loop(0, KCH, chunk, 0)
    plsc.subcore_barrier()
    for t in range(NZ):
        row = r0 + t * ZC
        g = si * NZ + t
        pltpu.sync_copy(acc_sh.at[pl.ds(row, ZC)], zb16)
        pltpu.sync_copy(zb16, acc_o.at[ci, g])
        pltpu.sync_copy(s_sh.at[pl.ds(row, ZC)], zb1)
        pltpu.sync_copy(zb1, s_o.at[ci, g])


_sc_call = functools.partial(
    pl.kernel,
    out_type=[
        jax.ShapeDtypeStruct((NC, GCH, ZC, C), jnp.float32),
        jax.ShapeDtypeStruct((NC, GCH, ZC), jnp.float32),
    ],
    mesh=plsc.VectorSubcoreMesh(core_axis_name="c", subcore_axis_name="s"),
    compiler_params=pltpu.CompilerParams(use_tc_tiling_on_sc=False),
    scratch_types=[
        pltpu.VMEM_SHARED((NT_PAD, C), jnp.float32),
        pltpu.VMEM_SHARED((NT_PAD,), jnp.float32),
        pltpu.VMEM((CHUNK,), jnp.int32),
        pltpu.VMEM((CHUNK,), jnp.int32),
        pltpu.VMEM((CHUNK,), jnp.float32),
        pltpu.VMEM((CHUNK,), jnp.float32),
        pltpu.VMEM((CHUNK,), jnp.float32),
        pltpu.VMEM((CHUNK, C), jnp.float32),
        pltpu.VMEM((CHUNK, C), jnp.float32),
        pltpu.VMEM((CHUNK,), jnp.float32),
        pltpu.VMEM((ZC, C), jnp.float32),
        pltpu.VMEM((ZC,), jnp.float32),
        pltpu.SemaphoreType.DMA,
    ],
)(_sc_body)


def _post_body(acc, s, res, g, b, out_o):
    accv = acc[...]
    sv = s[...]
    a = accv[0] + accv[1]
    ssum = sv[0] + sv[1]
    out = a / (ssum + 1e-16) + res[...]
    mu = jnp.mean(out, axis=1, keepdims=True)
    d = out - mu
    var = jnp.mean(d * d, axis=1, keepdims=True)
    y = d * lax.rsqrt(var + 1e-5) * g[...][None, :] + b[...][None, :]
    out_o[...] = jnp.maximum(y, 0.01 * y)


def _post_call(acc, s, res, ln_gamma, ln_beta):
    return pl.pallas_call(
        _post_body,
        grid=(GRID_N,),
        in_specs=[
            pl.BlockSpec((NC, BN, C), lambda i: (0, i, 0)),
            pl.BlockSpec((NC, BN, 1), lambda i: (0, i, 0)),
            pl.BlockSpec((BN, C), lambda i: (i, 0)),
            pl.BlockSpec((C,), lambda i: (0,)),
            pl.BlockSpec((C,), lambda i: (0,)),
        ],
        out_specs=pl.BlockSpec((BN, C), lambda i: (i, 0)),
        out_shape=jax.ShapeDtypeStruct((NT, C), jnp.float32),
    )(acc, s, res, ln_gamma, ln_beta)


def kernel(data_x, tasks_x, edge_index, edge_attr, W_src, W_dst, att_src,
           att_dst, W_edge, att_edge, res_W, bias, ln_gamma, ln_beta):
    att_src_v = att_src.reshape(C)
    att_dst_v = att_dst.reshape(C)
    att_edge_v = att_edge.reshape(C)

    xsrc, asrc3, adst3, res = _pre_call(
        data_x, tasks_x, W_src, W_dst, att_src_v, att_dst_v, res_W, bias)
    asrc = asrc3.reshape(ND)
    adst = adst3.reshape(NT)

    ei = jnp.pad(edge_index, ((0, 0), (0, E_PAD - E)))
    src = ei[0]
    dst = ei[1]
    eat3 = jnp.pad(edge_attr, ((0, E_PAD - E), (0, 0))).T.reshape(3, RB, EB_LANE)
    ae = _edge_call(eat3, W_edge, att_edge_v).reshape(E_PAD)

    z16 = jnp.zeros((ZC, C), jnp.float32)
    z1 = jnp.zeros((ZC,), jnp.float32)
    acc, s = _sc_call(src, dst, ae, xsrc, asrc, adst, z16, z1)

    return _post_call(acc.reshape(NC, NT_PAD, C), s.reshape(NC, NT_PAD, 1),
                      res, ln_gamma, ln_beta)


# speedup vs baseline: 76.0452x; 1.6510x over previous
import functools

import jax
import jax.numpy as jnp
from jax import lax
from jax.experimental import pallas as pl
from jax.experimental.pallas import tpu as pltpu
from jax.experimental.pallas import tpu_sc as plsc

ND = 100000
NT = 100000
E = 3200000
C = 16

NC = 2
NS = 16
NW = NC * NS
CHUNK = 128
KCH = 784
SJ = 4
EPT = KCH * CHUNK
E_PAD = NW * EPT
ECH = E_PAD // CHUNK
ROWS_PT = 6272
NT_PAD = NS * ROWS_PT
ZC = 128
NZ = ROWS_PT // ZC
GCH = NS * NZ

EB_LANE = 512
RB = E_PAD // EB_LANE
EB_ROWS = 128
GRID_B = RB // EB_ROWS

BN = 2000
GRID_N = NT // BN

NEG_BIG = -1e30


def _pre_body(dx, tx, Ws, Wd, asv, adv, Wr, b, xs_o, as_o, ad_o, res_o):
    dxv = dx[...]
    txv = tx[...]
    Wsv = Ws[...]
    xs = dxv[:, 0:1] * Wsv[0:1, :]
    for k in range(1, 5):
        xs = xs + dxv[:, k:k + 1] * Wsv[k:k + 1, :]
    xs_o[...] = xs
    as_o[...] = jnp.sum(xs * asv[...][None, :], axis=1).reshape(1, 1, BN)
    wdv = jnp.sum(Wd[...] * adv[...][None, :], axis=1)
    ad = txv[:, 0] * wdv[0]
    for k in range(1, 12):
        ad = ad + txv[:, k] * wdv[k]
    ad_o[...] = ad.reshape(1, 1, BN)
    Wrv = Wr[...]
    res = txv[:, 0:1] * Wrv[0:1, :]
    for k in range(1, 12):
        res = res + txv[:, k:k + 1] * Wrv[k:k + 1, :]
    res_o[...] = res + b[...][None, :]


def _pre_call(data_x, tasks_x, W_src, W_dst, att_src_v, att_dst_v, res_W, bias):
    return pl.pallas_call(
        _pre_body,
        grid=(GRID_N,),
        in_specs=[
            pl.BlockSpec((BN, 5), lambda i: (i, 0)),
            pl.BlockSpec((BN, 12), lambda i: (i, 0)),
            pl.BlockSpec((5, C), lambda i: (0, 0)),
            pl.BlockSpec((12, C), lambda i: (0, 0)),
            pl.BlockSpec((C,), lambda i: (0,)),
            pl.BlockSpec((C,), lambda i: (0,)),
            pl.BlockSpec((12, C), lambda i: (0, 0)),
            pl.BlockSpec((C,), lambda i: (0,)),
        ],
        out_specs=[
            pl.BlockSpec((BN, C), lambda i: (i, 0)),
            pl.BlockSpec((1, 1, BN), lambda i: (i, 0, 0)),
            pl.BlockSpec((1, 1, BN), lambda i: (i, 0, 0)),
            pl.BlockSpec((BN, C), lambda i: (i, 0)),
        ],
        out_shape=[
            jax.ShapeDtypeStruct((ND, C), jnp.float32),
            jax.ShapeDtypeStruct((GRID_N, 1, BN), jnp.float32),
            jax.ShapeDtypeStruct((GRID_N, 1, BN), jnp.float32),
            jax.ShapeDtypeStruct((NT, C), jnp.float32),
        ],
    )(data_x, tasks_x, W_src, W_dst, att_src_v, att_dst_v, res_W, bias)


def _edge_body(eat, We, aev, ae_o):
    w3 = jnp.sum(We[...] * aev[...][None, :], axis=1)
    b = eat[...]
    val = b[0] * w3[0] + b[1] * w3[1] + b[2] * w3[2]
    i = pl.program_id(0)
    rows = lax.broadcasted_iota(jnp.int32, (EB_ROWS, EB_LANE), 0)
    cols = lax.broadcasted_iota(jnp.int32, (EB_ROWS, EB_LANE), 1)
    flat = (i * EB_ROWS + rows) * EB_LANE + cols
    ae_o[...] = jnp.where(flat < E, val, NEG_BIG)


def _edge_call(eat3, W_edge, att_edge_v):
    return pl.pallas_call(
        _edge_body,
        grid=(GRID_B,),
        in_specs=[
            pl.BlockSpec((3, EB_ROWS, EB_LANE), lambda i: (0, i, 0)),
            pl.BlockSpec((3, C), lambda i: (0, 0)),
            pl.BlockSpec((C,), lambda i: (0,)),
        ],
        out_specs=pl.BlockSpec((EB_ROWS, EB_LANE), lambda i: (i, 0)),
        out_shape=jax.ShapeDtypeStruct((RB, EB_LANE), jnp.float32),
    )(eat3, W_edge, att_edge_v)


def _sc_body(src_h, dst_h, ae_h, xsrc_h, asrc_h, adst_h, z16_h, z1_h,
             acc_o, s_o,
             acc_sh, s_sh, srcv, dstv, aev, asv, adv, xs, msg, exb,
             zb16, zb1, gsem, ssem):
    ci = lax.axis_index("c")
    si = lax.axis_index("s")
    wid = ci * NS + si
    r0 = si * ROWS_PT
    pltpu.sync_copy(z16_h, zb16)
    pltpu.sync_copy(z1_h, zb1)
    for t in range(NZ):
        row = r0 + t * ZC
        pltpu.sync_copy(zb16, acc_sh.at[pl.ds(row, ZC)])
        pltpu.sync_copy(zb1, s_sh.at[pl.ds(row, ZC)])
    plsc.subcore_barrier()

    row_base0 = wid * (EPT // CHUNK)

    def superchunk(m, carry):
        row = row_base0 + m * SJ
        pltpu.sync_copy(src_h.at[pl.ds(row, SJ)], srcv)
        pltpu.sync_copy(dst_h.at[pl.ds(row, SJ)], dstv)
        pltpu.sync_copy(ae_h.at[pl.ds(row, SJ)], aev)
        gcp = []
        for j in range(SJ):
            gcp.append(pltpu.async_copy(xsrc_h.at[srcv.at[j]], xs.at[j], gsem))
            gcp.append(pltpu.async_copy(asrc_h.at[srcv.at[j]], asv.at[j], gsem))
            gcp.append(pltpu.async_copy(adst_h.at[dstv.at[j]], adv.at[j], gsem))
        scp = []
        for j in range(SJ):
            gcp[3 * j].wait()
            gcp[3 * j + 1].wait()
            gcp[3 * j + 2].wait()
            for g in range(CHUNK // 16):
                sl = pl.ds(g * 16, 16)
                al = asv[j, sl] + adv[j, sl] + aev[j, sl]
                al = jnp.maximum(al, al * 0.2)
                ex = jnp.exp(al)
                exb[j, sl] = ex
                for l in range(16):
                    i = g * 16 + l
                    msg[j, i, :] = xs[j, i, :] * ex[l]
            scp.append(pltpu.async_copy(exb.at[j], s_sh.at[dstv.at[j]], ssem,
                                        add=True))
            scp.append(pltpu.async_copy(msg.at[j], acc_sh.at[dstv.at[j]], ssem,
                                        add=True))
        for cp in scp:
            cp.wait()
        return carry

    lax.fori_loop(0, KCH // SJ, superchunk, 0)
    plsc.subcore_barrier()
    for t in range(NZ):
        row = r0 + t * ZC
        g = si * NZ + t
        pltpu.sync_copy(acc_sh.at[pl.ds(row, ZC)], zb16)
        pltpu.sync_copy(zb16, acc_o.at[ci, g])
        pltpu.sync_copy(s_sh.at[pl.ds(row, ZC)], zb1)
        pltpu.sync_copy(zb1, s_o.at[ci, g])


_sc_call = functools.partial(
    pl.kernel,
    out_type=[
        jax.ShapeDtypeStruct((NC, GCH, ZC, C), jnp.float32),
        jax.ShapeDtypeStruct((NC, GCH, ZC), jnp.float32),
    ],
    mesh=plsc.VectorSubcoreMesh(core_axis_name="c", subcore_axis_name="s"),
    compiler_params=pltpu.CompilerParams(use_tc_tiling_on_sc=False),
    scratch_types=[
        pltpu.VMEM_SHARED((NT_PAD, C), jnp.float32),
        pltpu.VMEM_SHARED((NT_PAD,), jnp.float32),
        pltpu.VMEM((SJ, CHUNK), jnp.int32),
        pltpu.VMEM((SJ, CHUNK), jnp.int32),
        pltpu.VMEM((SJ, CHUNK), jnp.float32),
        pltpu.VMEM((SJ, CHUNK), jnp.float32),
        pltpu.VMEM((SJ, CHUNK), jnp.float32),
        pltpu.VMEM((SJ, CHUNK, C), jnp.float32),
        pltpu.VMEM((SJ, CHUNK, C), jnp.float32),
        pltpu.VMEM((SJ, CHUNK), jnp.float32),
        pltpu.VMEM((ZC, C), jnp.float32),
        pltpu.VMEM((ZC,), jnp.float32),
        pltpu.SemaphoreType.DMA,
        pltpu.SemaphoreType.DMA,
    ],
)(_sc_body)


def _post_body(acc, s, res, g, b, out_o):
    accv = acc[...]
    sv = s[...]
    a = accv[0] + accv[1]
    ssum = sv[0] + sv[1]
    out = a / (ssum + 1e-16) + res[...]
    mu = jnp.mean(out, axis=1, keepdims=True)
    d = out - mu
    var = jnp.mean(d * d, axis=1, keepdims=True)
    y = d * lax.rsqrt(var + 1e-5) * g[...][None, :] + b[...][None, :]
    out_o[...] = jnp.maximum(y, 0.01 * y)


def _post_call(acc, s, res, ln_gamma, ln_beta):
    return pl.pallas_call(
        _post_body,
        grid=(GRID_N,),
        in_specs=[
            pl.BlockSpec((NC, BN, C), lambda i: (0, i, 0)),
            pl.BlockSpec((NC, BN, 1), lambda i: (0, i, 0)),
            pl.BlockSpec((BN, C), lambda i: (i, 0)),
            pl.BlockSpec((C,), lambda i: (0,)),
            pl.BlockSpec((C,), lambda i: (0,)),
        ],
        out_specs=pl.BlockSpec((BN, C), lambda i: (i, 0)),
        out_shape=jax.ShapeDtypeStruct((NT, C), jnp.float32),
    )(acc, s, res, ln_gamma, ln_beta)


def kernel(data_x, tasks_x, edge_index, edge_attr, W_src, W_dst, att_src,
           att_dst, W_edge, att_edge, res_W, bias, ln_gamma, ln_beta):
    att_src_v = att_src.reshape(C)
    att_dst_v = att_dst.reshape(C)
    att_edge_v = att_edge.reshape(C)

    xsrc, asrc3, adst3, res = _pre_call(
        data_x, tasks_x, W_src, W_dst, att_src_v, att_dst_v, res_W, bias)
    asrc = asrc3.reshape(ND)
    adst = adst3.reshape(NT)

    ei = jnp.pad(edge_index, ((0, 0), (0, E_PAD - E)))
    src = ei[0].reshape(ECH, CHUNK)
    dst = ei[1].reshape(ECH, CHUNK)
    eat3 = jnp.pad(edge_attr, ((0, E_PAD - E), (0, 0))).T.reshape(3, RB, EB_LANE)
    ae = _edge_call(eat3, W_edge, att_edge_v).reshape(ECH, CHUNK)

    z16 = jnp.zeros((ZC, C), jnp.float32)
    z1 = jnp.zeros((ZC,), jnp.float32)
    acc, s = _sc_call(src, dst, ae, xsrc, asrc, adst, z16, z1)

    return _post_call(acc.reshape(NC, NT_PAD, C), s.reshape(NC, NT_PAD, 1),
                      res, ln_gamma, ln_beta)


# speedup vs baseline: 98.1889x; 1.2912x over previous
import functools

import jax
import jax.numpy as jnp
from jax import lax
from jax.experimental import pallas as pl
from jax.experimental.pallas import tpu as pltpu
from jax.experimental.pallas import tpu_sc as plsc

ND = 100000
NT = 100000
E = 3200000
C = 16

NC = 2
NS = 16
NW = NC * NS
CHUNK = 128
KCH = 784
SJ = 4
EPT = KCH * CHUNK
E_PAD = NW * EPT
ECH = E_PAD // CHUNK
ROWS_PT = 6272
NT_PAD = NS * ROWS_PT
ZC = 128
NZ = ROWS_PT // ZC
GCH = NS * NZ

BR = 512
GRID_B = ECH // BR

BN = 2048
GRID_N = NT_PAD // BN
GC = BN // ZC

NEG_BIG = -1e30


def _pre_body(dx, tx, Ws, Wd, asv, adv, Wr, b, xs_o, as_o, ad_o, res_o):
    dxv = dx[...]
    txv = tx[...]
    xs = jnp.dot(dxv, Ws[...])
    xs_o[...] = xs
    as_o[...] = jnp.dot(xs, asv[...].reshape(C, 1)).reshape(BN)
    wdv = jnp.dot(Wd[...], adv[...].reshape(C, 1))
    ad_o[...] = jnp.dot(txv, wdv).reshape(BN)
    res_o[...] = jnp.dot(txv, Wr[...]) + b[...][None, :]


def _pre_call(data_x, tasks_x, W_src, W_dst, att_src_v, att_dst_v, res_W, bias):
    return pl.pallas_call(
        _pre_body,
        grid=(GRID_N,),
        in_specs=[
            pl.BlockSpec((BN, 5), lambda i: (i, 0)),
            pl.BlockSpec((BN, 12), lambda i: (i, 0)),
            pl.BlockSpec((5, C), lambda i: (0, 0)),
            pl.BlockSpec((12, C), lambda i: (0, 0)),
            pl.BlockSpec((C,), lambda i: (0,)),
            pl.BlockSpec((C,), lambda i: (0,)),
            pl.BlockSpec((12, C), lambda i: (0, 0)),
            pl.BlockSpec((C,), lambda i: (0,)),
        ],
        out_specs=[
            pl.BlockSpec((BN, C), lambda i: (i, 0)),
            pl.BlockSpec((BN,), lambda i: (i,)),
            pl.BlockSpec((BN,), lambda i: (i,)),
            pl.BlockSpec((BN, C), lambda i: (i, 0)),
        ],
        out_shape=[
            jax.ShapeDtypeStruct((NT_PAD, C), jnp.float32),
            jax.ShapeDtypeStruct((NT_PAD,), jnp.float32),
            jax.ShapeDtypeStruct((NT_PAD,), jnp.float32),
            jax.ShapeDtypeStruct((NT_PAD, C), jnp.float32),
        ],
    )(data_x, tasks_x, W_src, W_dst, att_src_v, att_dst_v, res_W, bias)


def _edge_body(eat, We, aev, ae_o):
    w3 = jnp.dot(We[...], aev[...].reshape(C, 1))
    b = eat[...]
    val = b[0] * w3[0, 0] + b[1] * w3[1, 0] + b[2] * w3[2, 0]
    i = pl.program_id(0)
    rows = lax.broadcasted_iota(jnp.int32, (BR, CHUNK), 0)
    cols = lax.broadcasted_iota(jnp.int32, (BR, CHUNK), 1)
    flat = (i * BR + rows) * CHUNK + cols
    ae_o[...] = jnp.where(flat < E, val, NEG_BIG)


def _edge_call(eat3, W_edge, att_edge_v):
    return pl.pallas_call(
        _edge_body,
        grid=(GRID_B,),
        in_specs=[
            pl.BlockSpec((3, BR, CHUNK), lambda i: (0, i, 0)),
            pl.BlockSpec((3, C), lambda i: (0, 0)),
            pl.BlockSpec((C,), lambda i: (0,)),
        ],
        out_specs=pl.BlockSpec((BR, CHUNK), lambda i: (i, 0)),
        out_shape=jax.ShapeDtypeStruct((ECH, CHUNK), jnp.float32),
    )(eat3, W_edge, att_edge_v)


def _sc_body(ei_h, ae_h, xsrc_h, asrc_h, adst_h, z16_h, z1_h,
             acc_o, s_o,
             acc_sh, s_sh, srcv, dstv, aev, asv, adv, xs, msg, exb,
             zb16, zb1, gsem, ssem):
    ci = lax.axis_index("c")
    si = lax.axis_index("s")
    wid = ci * NS + si
    r0 = si * ROWS_PT
    pltpu.sync_copy(z16_h, zb16)
    pltpu.sync_copy(z1_h, zb1)
    for t in range(NZ):
        row = r0 + t * ZC
        pltpu.sync_copy(zb16, acc_sh.at[pl.ds(row, ZC)])
        pltpu.sync_copy(zb1, s_sh.at[pl.ds(row, ZC)])
    plsc.subcore_barrier()

    row_base0 = wid * KCH

    def superchunk(m, carry):
        row = row_base0 + m * SJ
        pltpu.sync_copy(ei_h.at[0, pl.ds(row, SJ)], srcv)
        pltpu.sync_copy(ei_h.at[1, pl.ds(row, SJ)], dstv)
        pltpu.sync_copy(ae_h.at[pl.ds(row, SJ)], aev)
        gcp = []
        for j in range(SJ):
            gcp.append(pltpu.async_copy(xsrc_h.at[srcv.at[j]], xs.at[j], gsem))
            gcp.append(pltpu.async_copy(asrc_h.at[srcv.at[j]], asv.at[j], gsem))
            gcp.append(pltpu.async_copy(adst_h.at[dstv.at[j]], adv.at[j], gsem))
        scp = []
        for j in range(SJ):
            gcp[3 * j].wait()
            gcp[3 * j + 1].wait()
            gcp[3 * j + 2].wait()
            for g in range(CHUNK // 16):
                sl = pl.ds(g * 16, 16)
                al = asv[j, sl] + adv[j, sl] + aev[j, sl]
                al = jnp.maximum(al, al * 0.2)
                ex = jnp.exp(al)
                exb[j, sl] = ex
                for l in range(16):
                    i = g * 16 + l
                    msg[j, i, :] = xs[j, i, :] * ex[l]
            scp.append(pltpu.async_copy(exb.at[j], s_sh.at[dstv.at[j]], ssem,
                                        add=True))
            scp.append(pltpu.async_copy(msg.at[j], acc_sh.at[dstv.at[j]], ssem,
                                        add=True))
        for cp in scp:
            cp.wait()
        return carry

    lax.fori_loop(0, KCH // SJ, superchunk, 0)
    plsc.subcore_barrier()
    for t in range(NZ):
        row = r0 + t * ZC
        g = si * NZ + t
        pltpu.sync_copy(acc_sh.at[pl.ds(row, ZC)], zb16)
        pltpu.sync_copy(zb16, acc_o.at[ci, g])
        pltpu.sync_copy(s_sh.at[pl.ds(row, ZC)], zb1)
        pltpu.sync_copy(zb1, s_o.at[ci, g])


_sc_call = functools.partial(
    pl.kernel,
    out_type=[
        jax.ShapeDtypeStruct((NC, GCH, ZC, C), jnp.float32),
        jax.ShapeDtypeStruct((NC, GCH, ZC), jnp.float32),
    ],
    mesh=plsc.VectorSubcoreMesh(core_axis_name="c", subcore_axis_name="s"),
    compiler_params=pltpu.CompilerParams(use_tc_tiling_on_sc=False),
    scratch_types=[
        pltpu.VMEM_SHARED((NT_PAD, C), jnp.float32),
        pltpu.VMEM_SHARED((NT_PAD,), jnp.float32),
        pltpu.VMEM((SJ, CHUNK), jnp.int32),
        pltpu.VMEM((SJ, CHUNK), jnp.int32),
        pltpu.VMEM((SJ, CHUNK), jnp.float32),
        pltpu.VMEM((SJ, CHUNK), jnp.float32),
        pltpu.VMEM((SJ, CHUNK), jnp.float32),
        pltpu.VMEM((SJ, CHUNK, C), jnp.float32),
        pltpu.VMEM((SJ, CHUNK, C), jnp.float32),
        pltpu.VMEM((SJ, CHUNK), jnp.float32),
        pltpu.VMEM((ZC, C), jnp.float32),
        pltpu.VMEM((ZC,), jnp.float32),
        pltpu.SemaphoreType.DMA,
        pltpu.SemaphoreType.DMA,
    ],
)(_sc_body)


def _post_body(acc, s, res, g, b, out_o):
    accv = acc[...]
    sv = s[...]
    a = accv[0] + accv[1]
    ssum = sv[0] + sv[1]
    resv = res[...].reshape(GC, ZC, C)
    out = a / (ssum[:, :, None] + 1e-16) + resv
    mu = jnp.mean(out, axis=2, keepdims=True)
    d = out - mu
    var = jnp.mean(d * d, axis=2, keepdims=True)
    y = d * lax.rsqrt(var + 1e-5) * g[...][None, None, :] + b[...][None, None, :]
    out_o[...] = jnp.maximum(y, 0.01 * y).reshape(BN, C)


def _post_call(acc, s, res, ln_gamma, ln_beta):
    return pl.pallas_call(
        _post_body,
        grid=(GRID_N,),
        in_specs=[
            pl.BlockSpec((NC, GC, ZC, C), lambda i: (0, i, 0, 0)),
            pl.BlockSpec((NC, GC, ZC), lambda i: (0, i, 0)),
            pl.BlockSpec((BN, C), lambda i: (i, 0)),
            pl.BlockSpec((C,), lambda i: (0,)),
            pl.BlockSpec((C,), lambda i: (0,)),
        ],
        out_specs=pl.BlockSpec((BN, C), lambda i: (i, 0)),
        out_shape=jax.ShapeDtypeStruct((NT_PAD, C), jnp.float32),
    )(acc, s, res, ln_gamma, ln_beta)


def kernel(data_x, tasks_x, edge_index, edge_attr, W_src, W_dst, att_src,
           att_dst, W_edge, att_edge, res_W, bias, ln_gamma, ln_beta):
    att_src_v = att_src.reshape(C)
    att_dst_v = att_dst.reshape(C)
    att_edge_v = att_edge.reshape(C)

    dxp = jnp.pad(data_x, ((0, NT_PAD - ND), (0, 0)))
    txp = jnp.pad(tasks_x, ((0, NT_PAD - NT), (0, 0)))
    xsrc, asrc, adst, res = _pre_call(
        dxp, txp, W_src, W_dst, att_src_v, att_dst_v, res_W, bias)

    ei3 = jnp.pad(edge_index, ((0, 0), (0, E_PAD - E))).reshape(2, ECH, CHUNK)
    eat3 = jnp.pad(edge_attr, ((0, E_PAD - E), (0, 0))).T.reshape(3, ECH, CHUNK)
    ae = _edge_call(eat3, W_edge, att_edge_v)

    z16 = jnp.zeros((ZC, C), jnp.float32)
    z1 = jnp.zeros((ZC,), jnp.float32)
    acc, s = _sc_call(ei3, ae, xsrc, asrc, adst, z16, z1)

    return _post_call(acc, s, res, ln_gamma, ln_beta)[:NT]


# speedup vs baseline: 138.1044x; 1.4065x over previous
import functools

import jax
import jax.numpy as jnp
from jax import lax
from jax.experimental import pallas as pl
from jax.experimental.pallas import tpu as pltpu
from jax.experimental.pallas import tpu_sc as plsc

ND = 100000
NT = 100000
E = 3200000
C = 16

NC = 2
NS = 16
NW = NC * NS
CHUNK = 128
KCH = 784
SJ = 2
NH = KCH // SJ
EPT = KCH * CHUNK
E_PAD = NW * EPT
ECH = E_PAD // CHUNK
ROWS_PT = 6272
NT_PAD = NS * ROWS_PT
ZC = 128
NZ = ROWS_PT // ZC
GCH = NS * NZ

BR = 512
GRID_B = ECH // BR

BN = 2048
GRID_N = NT_PAD // BN
GC = BN // ZC

NEG_BIG = -1e30


def _pre_body(dx, tx, Ws, Wd, asv, adv, Wr, b, xs_o, as_o, ad_o, res_o):
    dxv = dx[...]
    txv = tx[...]
    xs = jnp.dot(dxv, Ws[...])
    xs_o[...] = xs
    as_o[...] = jnp.dot(xs, asv[...].reshape(C, 1)).reshape(BN)
    wdv = jnp.dot(Wd[...], adv[...].reshape(C, 1))
    ad_o[...] = jnp.dot(txv, wdv).reshape(BN)
    res_o[...] = jnp.dot(txv, Wr[...]) + b[...][None, :]


def _pre_call(data_x, tasks_x, W_src, W_dst, att_src_v, att_dst_v, res_W, bias):
    return pl.pallas_call(
        _pre_body,
        grid=(GRID_N,),
        in_specs=[
            pl.BlockSpec((BN, 5), lambda i: (i, 0)),
            pl.BlockSpec((BN, 12), lambda i: (i, 0)),
            pl.BlockSpec((5, C), lambda i: (0, 0)),
            pl.BlockSpec((12, C), lambda i: (0, 0)),
            pl.BlockSpec((C,), lambda i: (0,)),
            pl.BlockSpec((C,), lambda i: (0,)),
            pl.BlockSpec((12, C), lambda i: (0, 0)),
            pl.BlockSpec((C,), lambda i: (0,)),
        ],
        out_specs=[
            pl.BlockSpec((BN, C), lambda i: (i, 0)),
            pl.BlockSpec((BN,), lambda i: (i,)),
            pl.BlockSpec((BN,), lambda i: (i,)),
            pl.BlockSpec((BN, C), lambda i: (i, 0)),
        ],
        out_shape=[
            jax.ShapeDtypeStruct((NT_PAD, C), jnp.float32),
            jax.ShapeDtypeStruct((NT_PAD,), jnp.float32),
            jax.ShapeDtypeStruct((NT_PAD,), jnp.float32),
            jax.ShapeDtypeStruct((NT_PAD, C), jnp.float32),
        ],
    )(data_x, tasks_x, W_src, W_dst, att_src_v, att_dst_v, res_W, bias)


def _edge_body(eat, We, aev, ae_o):
    w3 = jnp.dot(We[...], aev[...].reshape(C, 1))
    b = eat[...]
    val = b[0] * w3[0, 0] + b[1] * w3[1, 0] + b[2] * w3[2, 0]
    i = pl.program_id(0)
    rows = lax.broadcasted_iota(jnp.int32, (BR, CHUNK), 0)
    cols = lax.broadcasted_iota(jnp.int32, (BR, CHUNK), 1)
    flat = (i * BR + rows) * CHUNK + cols
    ae_o[...] = jnp.where(flat < E, val, NEG_BIG)


def _edge_call(eat3, W_edge, att_edge_v):
    return pl.pallas_call(
        _edge_body,
        grid=(GRID_B,),
        in_specs=[
            pl.BlockSpec((3, BR, CHUNK), lambda i: (0, i, 0)),
            pl.BlockSpec((3, C), lambda i: (0, 0)),
            pl.BlockSpec((C,), lambda i: (0,)),
        ],
        out_specs=pl.BlockSpec((BR, CHUNK), lambda i: (i, 0)),
        out_shape=jax.ShapeDtypeStruct((ECH, CHUNK), jnp.float32),
    )(eat3, W_edge, att_edge_v)


def _sc_body(ei_h, ae_h, xsrc_h, asrc_h, adst_h, z16_h, z1_h,
             acc_o, s_o,
             acc_sh, s_sh,
             srcv0, dstv0, aev0, asv0, adv0, xs0, msg0, exb0, dsts0,
             srcv1, dstv1, aev1, asv1, adv1, xs1, msg1, exb1, dsts1,
             zb16, zb1, msem, gsem0, gsem1, ssem0, ssem1):
    ci = lax.axis_index("c")
    si = lax.axis_index("s")
    wid = ci * NS + si
    r0 = si * ROWS_PT
    pltpu.sync_copy(z16_h, zb16)
    pltpu.sync_copy(z1_h, zb1)
    for t in range(NZ):
        row = r0 + t * ZC
        pltpu.sync_copy(zb16, acc_sh.at[pl.ds(row, ZC)])
        pltpu.sync_copy(zb1, s_sh.at[pl.ds(row, ZC)])
    plsc.subcore_barrier()

    row_base0 = wid * KCH
    sets = ((srcv0, dstv0, aev0, asv0, adv0, xs0, msg0, exb0, dsts0,
             gsem0, ssem0),
            (srcv1, dstv1, aev1, asv1, adv1, xs1, msg1, exb1, dsts1,
             gsem1, ssem1))

    def meta_row(m):
        return row_base0 + m * SJ

    def fire_meta(m, st):
        srcv, dstv, aev = st[0], st[1], st[2]
        row = meta_row(m)
        pltpu.async_copy(ei_h.at[0, pl.ds(row, SJ)], srcv, msem)
        pltpu.async_copy(ei_h.at[1, pl.ds(row, SJ)], dstv, msem)
        pltpu.async_copy(ae_h.at[pl.ds(row, SJ)], aev, msem)

    def wait_meta(st):
        srcv, dstv, aev = st[0], st[1], st[2]
        row = meta_row(0)
        pltpu.make_async_copy(ei_h.at[0, pl.ds(row, SJ)], srcv, msem).wait()
        pltpu.make_async_copy(ei_h.at[1, pl.ds(row, SJ)], dstv, msem).wait()
        pltpu.make_async_copy(ae_h.at[pl.ds(row, SJ)], aev, msem).wait()

    def fire_gathers(st):
        srcv, dstv, asv, adv, xs = st[0], st[1], st[3], st[4], st[5]
        gsem = st[9]
        for j in range(SJ):
            pltpu.async_copy(xsrc_h.at[srcv.at[j]], xs.at[j], gsem)
            pltpu.async_copy(asrc_h.at[srcv.at[j]], asv.at[j], gsem)
            pltpu.async_copy(adst_h.at[dstv.at[j]], adv.at[j], gsem)

    def wait_gathers(st):
        srcv, dstv, asv, adv, xs = st[0], st[1], st[3], st[4], st[5]
        gsem = st[9]
        for j in range(SJ):
            pltpu.make_async_copy(xsrc_h.at[srcv.at[j]], xs.at[j], gsem).wait()
            pltpu.make_async_copy(asrc_h.at[srcv.at[j]], asv.at[j], gsem).wait()
            pltpu.make_async_copy(adst_h.at[dstv.at[j]], adv.at[j], gsem).wait()

    def fire_scatters(st):
        msg, exb, dsts, ssem = st[6], st[7], st[8], st[10]
        for j in range(SJ):
            pltpu.async_copy(exb.at[j], s_sh.at[dsts.at[j]], ssem, add=True)
            pltpu.async_copy(msg.at[j], acc_sh.at[dsts.at[j]], ssem, add=True)

    def wait_scatters(st):
        msg, exb, dsts, ssem = st[6], st[7], st[8], st[10]
        for j in range(SJ):
            pltpu.make_async_copy(exb.at[j], s_sh.at[dsts.at[j]], ssem).wait()
            pltpu.make_async_copy(msg.at[j], acc_sh.at[dsts.at[j]], ssem).wait()

    def compute(st):
        dstv, aev, asv, adv, xs, msg, exb, dsts = (
            st[1], st[2], st[3], st[4], st[5], st[6], st[7], st[8])
        for j in range(SJ):
            for g in range(CHUNK // 16):
                sl = pl.ds(g * 16, 16)
                dsts[j, sl] = dstv[j, sl]
                al = asv[j, sl] + adv[j, sl] + aev[j, sl]
                al = jnp.maximum(al, al * 0.2)
                ex = jnp.exp(al)
                exb[j, sl] = ex
                for l in range(16):
                    i = g * 16 + l
                    msg[j, i, :] = xs[j, i, :] * ex[l]

    fire_meta(0, sets[0])
    wait_meta(sets[0])
    fire_gathers(sets[0])
    fire_meta(1, sets[1])

    def pair(t, carry):
        for p in range(2):
            m = 2 * t + p
            st = sets[p]
            ot = sets[1 - p]

            @pl.when(m + 1 < NH)
            def _():
                wait_meta(ot)
                fire_gathers(ot)

            @pl.when(m >= 2)
            def _():
                wait_scatters(st)

            wait_gathers(st)
            compute(st)
            fire_scatters(st)

            @pl.when(m + 2 < NH)
            def _():
                fire_meta(m + 2, st)
        return carry

    lax.fori_loop(0, NH // 2, pair, 0)
    wait_scatters(sets[0])
    wait_scatters(sets[1])
    plsc.subcore_barrier()
    for t in range(NZ):
        row = r0 + t * ZC
        g = si * NZ + t
        pltpu.sync_copy(acc_sh.at[pl.ds(row, ZC)], zb16)
        pltpu.sync_copy(zb16, acc_o.at[ci, g])
        pltpu.sync_copy(s_sh.at[pl.ds(row, ZC)], zb1)
        pltpu.sync_copy(zb1, s_o.at[ci, g])


_sc_call = functools.partial(
    pl.kernel,
    out_type=[
        jax.ShapeDtypeStruct((NC, GCH, ZC, C), jnp.float32),
        jax.ShapeDtypeStruct((NC, GCH, ZC), jnp.float32),
    ],
    mesh=plsc.VectorSubcoreMesh(core_axis_name="c", subcore_axis_name="s"),
    compiler_params=pltpu.CompilerParams(use_tc_tiling_on_sc=False),
    scratch_types=(
        [pltpu.VMEM_SHARED((NT_PAD, C), jnp.float32),
         pltpu.VMEM_SHARED((NT_PAD,), jnp.float32)]
        + 2 * [pltpu.VMEM((SJ, CHUNK), jnp.int32),
               pltpu.VMEM((SJ, CHUNK), jnp.int32),
               pltpu.VMEM((SJ, CHUNK), jnp.float32),
               pltpu.VMEM((SJ, CHUNK), jnp.float32),
               pltpu.VMEM((SJ, CHUNK), jnp.float32),
               pltpu.VMEM((SJ, CHUNK, C), jnp.float32),
               pltpu.VMEM((SJ, CHUNK, C), jnp.float32),
               pltpu.VMEM((SJ, CHUNK), jnp.float32),
               pltpu.VMEM((SJ, CHUNK), jnp.int32)]
        + [pltpu.VMEM((ZC, C), jnp.float32),
           pltpu.VMEM((ZC,), jnp.float32),
           pltpu.SemaphoreType.DMA,
           pltpu.SemaphoreType.DMA,
           pltpu.SemaphoreType.DMA,
           pltpu.SemaphoreType.DMA,
           pltpu.SemaphoreType.DMA]
    ),
)(_sc_body)


def _post_body(acc, s, res, g, b, out_o):
    accv = acc[...]
    sv = s[...]
    a = accv[0] + accv[1]
    ssum = sv[0] + sv[1]
    resv = res[...].reshape(GC, ZC, C)
    out = a / (ssum[:, :, None] + 1e-16) + resv
    mu = jnp.mean(out, axis=2, keepdims=True)
    d = out - mu
    var = jnp.mean(d * d, axis=2, keepdims=True)
    y = d * lax.rsqrt(var + 1e-5) * g[...][None, None, :] + b[...][None, None, :]
    out_o[...] = jnp.maximum(y, 0.01 * y).reshape(BN, C)


def _post_call(acc, s, res, ln_gamma, ln_beta):
    return pl.pallas_call(
        _post_body,
        grid=(GRID_N,),
        in_specs=[
            pl.BlockSpec((NC, GC, ZC, C), lambda i: (0, i, 0, 0)),
            pl.BlockSpec((NC, GC, ZC), lambda i: (0, i, 0)),
            pl.BlockSpec((BN, C), lambda i: (i, 0)),
            pl.BlockSpec((C,), lambda i: (0,)),
            pl.BlockSpec((C,), lambda i: (0,)),
        ],
        out_specs=pl.BlockSpec((BN, C), lambda i: (i, 0)),
        out_shape=jax.ShapeDtypeStruct((NT_PAD, C), jnp.float32),
    )(acc, s, res, ln_gamma, ln_beta)


def kernel(data_x, tasks_x, edge_index, edge_attr, W_src, W_dst, att_src,
           att_dst, W_edge, att_edge, res_W, bias, ln_gamma, ln_beta):
    att_src_v = att_src.reshape(C)
    att_dst_v = att_dst.reshape(C)
    att_edge_v = att_edge.reshape(C)

    xsrc, asrc, adst, res = _pre_call(
        data_x, tasks_x, W_src, W_dst, att_src_v, att_dst_v, res_W, bias)

    ei3 = jnp.pad(edge_index, ((0, 0), (0, E_PAD - E))).reshape(2, ECH, CHUNK)
    eat3 = jnp.pad(edge_attr, ((0, E_PAD - E), (0, 0))).T.reshape(3, ECH, CHUNK)
    ae = _edge_call(eat3, W_edge, att_edge_v)

    z16 = jnp.zeros((ZC, C), jnp.float32)
    z1 = jnp.zeros((ZC,), jnp.float32)
    acc, s = _sc_call(ei3, ae, xsrc, asrc, adst, z16, z1)

    return _post_call(acc, s, res, ln_gamma, ln_beta)[:NT]
